# baseline (speedup 1.0000x reference)
import functools

import jax
import jax.numpy as jnp
from jax import lax
from jax.experimental import pallas as pl
from jax.experimental.pallas import tpu as pltpu
from jax.experimental.pallas import tpu_sc as plsc

BATCH = 16384
DIM = 16

_info = plsc.get_sparse_core_info()
_NC, _NS = _info.num_cores, _info.num_subcores
_NW = _NC * _NS
_BPW = BATCH // _NW
_CHUNK = 128
_NCHUNK = _BPW // _CHUNK

_mesh = plsc.VectorSubcoreMesh(core_axis_name="c", subcore_axis_name="s")


@functools.partial(
    pl.kernel,
    mesh=_mesh,
    compiler_params=pltpu.CompilerParams(use_tc_tiling_on_sc=False),
    out_type=[
        jax.ShapeDtypeStruct((BATCH, DIM), jnp.float32),
        jax.ShapeDtypeStruct((BATCH, DIM), jnp.float32),
    ],
    scratch_types=[
        pltpu.VMEM((_BPW,), jnp.int32),
        pltpu.VMEM((_BPW,), jnp.int32),
        pltpu.VMEM((_BPW, DIM), jnp.float32),
        pltpu.VMEM((_BPW, DIM), jnp.float32),
        pltpu.SemaphoreType.DMA,
    ],
)
def _sc_gather(user_hbm, item_hbm, utab_hbm, itab_hbm, ue_out, ie_out,
               uidx_v, iidx_v, urows_v, irows_v, sem):
    wid = lax.axis_index("s") * _NC + lax.axis_index("c")
    base = wid * _BPW
    pltpu.sync_copy(user_hbm.at[pl.ds(base, _BPW)], uidx_v)
    pltpu.sync_copy(item_hbm.at[pl.ds(base, _BPW)], iidx_v)
    copies = []
    for c in range(_NCHUNK):
        copies.append(pltpu.async_copy(
            utab_hbm.at[uidx_v.at[pl.ds(c * _CHUNK, _CHUNK)]],
            urows_v.at[pl.ds(c * _CHUNK, _CHUNK)], sem))
        copies.append(pltpu.async_copy(
            itab_hbm.at[iidx_v.at[pl.ds(c * _CHUNK, _CHUNK)]],
            irows_v.at[pl.ds(c * _CHUNK, _CHUNK)], sem))
    for cp in copies:
        cp.wait()
    pltpu.sync_copy(urows_v, ue_out.at[pl.ds(base, _BPW)])
    pltpu.sync_copy(irows_v, ie_out.at[pl.ds(base, _BPW)])


def _tc_body(ue_ref, ie_ref, w_ref, out_ref):
    svd = jnp.dot(ue_ref[...], w_ref[...], preferred_element_type=jnp.float32)
    out_ref[...] = jnp.sum(svd * ie_ref[...], axis=1)


def _tc_compute(ue, ie, w):
    return pl.pallas_call(
        _tc_body,
        out_shape=jax.ShapeDtypeStruct((BATCH,), jnp.float32),
    )(ue, ie, w)


@jax.jit
def kernel(user, item, user_table, item_table, svd_weight):
    user = user.astype(jnp.int32)
    item = item.astype(jnp.int32)
    ue, ie = _sc_gather(user, item, user_table, item_table)
    return _tc_compute(ue, ie, svd_weight)


# speedup vs baseline: 1.0081x; 1.0081x over previous
import functools

import jax
import jax.numpy as jnp
from jax import lax
from jax.experimental import pallas as pl
from jax.experimental.pallas import tpu as pltpu
from jax.experimental.pallas import tpu_sc as plsc

BATCH = 16384
DIM = 16
NROWS = 1000000
PACK = 8
TAB8 = NROWS // PACK

_info = plsc.get_sparse_core_info()
_NC, _NS = _info.num_cores, _info.num_subcores
_NW = _NC * _NS
_BPW = BATCH // _NW
_CHUNK = 128
_NCHUNK = _BPW // _CHUNK
_NBLK = _CHUNK // DIM

_mesh = plsc.VectorSubcoreMesh(core_axis_name="c", subcore_axis_name="s")


@functools.partial(
    pl.kernel,
    mesh=_mesh,
    compiler_params=pltpu.CompilerParams(
        use_tc_tiling_on_sc=False, needs_layout_passes=False),
    out_type=jax.ShapeDtypeStruct((BATCH,), jnp.float32),
    scratch_types=[
        pltpu.VMEM((_BPW,), jnp.int32),
        pltpu.VMEM((_BPW,), jnp.int32),
        pltpu.VMEM((_BPW,), jnp.int32),
        pltpu.VMEM((_BPW,), jnp.int32),
        pltpu.VMEM((DIM * DIM * 16,), jnp.float32),
        pltpu.VMEM((_CHUNK, 128), jnp.float32),
        pltpu.VMEM((_CHUNK, 128), jnp.float32),
        pltpu.VMEM((_BPW,), jnp.float32),
        pltpu.SemaphoreType.DMA,
    ],
)
def _sc_svd(user_hbm, item_hbm, utab_hbm, itab_hbm, wsplat_hbm, out_hbm,
            uidx_v, iidx_v, u8_v, i8_v, w_v, urow_v, irow_v, out_v, sem):
    wid = lax.axis_index("s") * _NC + lax.axis_index("c")
    base = wid * _BPW
    pltpu.sync_copy(user_hbm.at[pl.ds(base, _BPW)], uidx_v)
    pltpu.sync_copy(item_hbm.at[pl.ds(base, _BPW)], iidx_v)
    pltpu.sync_copy(wsplat_hbm, w_v)

    @pl.loop(0, _BPW // 16)
    def _rowids(v):
        s = pl.ds(v * 16, 16)
        u8_v[s] = lax.shift_right_logical(uidx_v[s], 3)
        i8_v[s] = lax.shift_right_logical(iidx_v[s], 3)

    iota = lax.broadcasted_iota(jnp.int32, (16,), 0)

    for c in range(_NCHUNK):
        ucp = pltpu.async_copy(
            utab_hbm.at[u8_v.at[pl.ds(c * _CHUNK, _CHUNK)]], urow_v, sem)
        icp = pltpu.async_copy(
            itab_hbm.at[i8_v.at[pl.ds(c * _CHUNK, _CHUNK)]], irow_v, sem)
        ucp.wait()
        icp.wait()

        @pl.loop(0, _NBLK)
        def _block(b):
            off = c * _CHUNK + b * 16
            rowid = iota + b * 16
            uvec = uidx_v[pl.ds(off, 16)]
            ivec = iidx_v[pl.ds(off, 16)]
            mu = (uvec & 7) << 4
            mi = (ivec & 7) << 4
            ucols = [plsc.load_gather(urow_v, [rowid, mu + j])
                     for j in range(DIM)]
            icols = [plsc.load_gather(irow_v, [rowid, mi + j])
                     for j in range(DIM)]
            acc = jnp.zeros((16,), jnp.float32)
            for j in range(DIM):
                s = w_v[pl.ds(j * 16, 16)] * ucols[0]
                for k in range(1, DIM):
                    s = s + w_v[pl.ds((k * DIM + j) * 16, 16)] * ucols[k]
                acc = acc + s * icols[j]
            out_v[pl.ds(off, 16)] = acc

    pltpu.sync_copy(out_v, out_hbm.at[pl.ds(base, _BPW)])


@jax.jit
def kernel(user, item, user_table, item_table, svd_weight):
    user = user.astype(jnp.int32)
    item = item.astype(jnp.int32)
    ut8 = user_table.reshape(TAB8, PACK * DIM)
    it8 = item_table.reshape(TAB8, PACK * DIM)
    wsplat = jnp.repeat(svd_weight.reshape(DIM * DIM), 16)
    return _sc_svd(user, item, ut8, it8, wsplat)


# speedup vs baseline: 5.8444x; 5.7977x over previous
import functools

import jax
import jax.numpy as jnp
from jax import lax
from jax.experimental import pallas as pl
from jax.experimental.pallas import tpu as pltpu
from jax.experimental.pallas import tpu_sc as plsc

BATCH = 16384
DIM = 16
NROWS = 1000000

_info = plsc.get_sparse_core_info()
_NC, _NS = _info.num_cores, _info.num_subcores
_NW = _NC * _NS
_BPW = BATCH // _NW
_CHUNK = 16
_NCHUNK = _BPW // _CHUNK

_mesh = plsc.VectorSubcoreMesh(core_axis_name="c", subcore_axis_name="s")


@functools.partial(
    pl.kernel,
    mesh=_mesh,
    compiler_params=pltpu.CompilerParams(needs_layout_passes=False),
    out_type=jax.ShapeDtypeStruct((BATCH,), jnp.float32),
    scratch_types=[
        pltpu.VMEM((_BPW,), jnp.int32),
        pltpu.VMEM((_BPW,), jnp.int32),
        pltpu.VMEM((DIM * DIM * 16,), jnp.float32),
        pltpu.VMEM((2, 8, _CHUNK * 128), jnp.float32),
        pltpu.VMEM((2, 8, _CHUNK * 128), jnp.float32),
        pltpu.VMEM((_BPW,), jnp.float32),
        pltpu.SemaphoreType.DMA,
    ],
)
def _sc_svd(user_hbm, item_hbm, utab_hbm, itab_hbm, wsplat_hbm, out_hbm,
            uidx_v, iidx_v, w_v, uf_v, if_v, out_v, sem):
    wid = lax.axis_index("s") * _NC + lax.axis_index("c")
    base = wid * _BPW
    pltpu.sync_copy(user_hbm.at[pl.ds(base, _BPW)], uidx_v)
    pltpu.sync_copy(item_hbm.at[pl.ds(base, _BPW)], iidx_v)
    pltpu.sync_copy(wsplat_hbm, w_v)

    iota = lax.broadcasted_iota(jnp.int32, (16,), 0)

    @pl.loop(0, _NCHUNK)
    def _chunk(c):
        uvec_c = uidx_v[pl.ds(c * _CHUNK, 16)]
        ivec_c = iidx_v[pl.ds(c * _CHUNK, 16)]

        @pl.loop(0, _CHUNK)
        def _fire(e):
            d128 = pl.ds(e * 128, 128)
            u = jnp.sum(jnp.where(iota == e, uvec_c, 0))
            i = jnp.sum(jnp.where(iota == e, ivec_c, 0))
            us = pl.ds(pl.multiple_of((u >> 7) * 128, 128), 128)
            isl = pl.ds(pl.multiple_of((i >> 7) * 128, 128), 128)
            for tr in range(2):
                t1 = pl.ds(tr, 1)
                pltpu.async_copy(utab_hbm.at[t1, :, us],
                                 uf_v.at[t1, :, d128], sem)
                pltpu.async_copy(itab_hbm.at[t1, :, isl],
                                 if_v.at[t1, :, d128], sem)

        for buf in (uf_v, if_v):
            pltpu.make_async_copy(
                utab_hbm.at[:, :, pl.ds(0, _CHUNK * 128)], buf, sem).wait()

        s16 = pl.ds(c * _CHUNK, 16)
        uvec = uidx_v[s16]
        ivec = iidx_v[s16]
        upos = iota * 128 + (uvec & 127)
        ipos = iota * 128 + (ivec & 127)
        zero = iota * 0
        ucols = [plsc.load_gather(uf_v, [zero + k // 8, zero + k % 8, upos])
                 for k in range(DIM)]
        icols = [plsc.load_gather(if_v, [zero + k // 8, zero + k % 8, ipos])
                 for k in range(DIM)]
        acc = jnp.zeros((16,), jnp.float32)
        for k in range(DIM):
            s = w_v[pl.ds(k * DIM * 16, 16)] * icols[0]
            for j in range(1, DIM):
                s = s + w_v[pl.ds((k * DIM + j) * 16, 16)] * icols[j]
            acc = acc + ucols[k] * s
        out_v[s16] = acc

    pltpu.sync_copy(out_v, out_hbm.at[pl.ds(base, _BPW)])


@jax.jit
def kernel(user, item, user_table, item_table, svd_weight):
    user = user.astype(jnp.int32)
    item = item.astype(jnp.int32)
    ut3 = user_table.T.reshape(2, 8, NROWS)
    it3 = item_table.T.reshape(2, 8, NROWS)
    wsplat = jnp.repeat(svd_weight.reshape(DIM * DIM), 16)
    return _sc_svd(user, item, ut3, it3, wsplat)
